# baseline (speedup 1.0000x reference)
import functools

import jax
import jax.numpy as jnp
from jax import lax
from jax.experimental import pallas as pl
from jax.experimental.pallas import tpu as pltpu
from jax.experimental.pallas import tpu_sc as plsc

N = 10000
D = 128
E = 320000

NC = 2
NS = 16
NW = NC * NS
EPW = E // NW
K = 80
NCH = EPW // K
ZW = 10
ZRPT = N // ZW

_mesh = plsc.VectorSubcoreMesh(core_axis_name="c", subcore_axis_name="s")
_sc_params = pltpu.CompilerParams(needs_layout_passes=False)
_f32 = jnp.float32
_i32 = jnp.int32


NPAD = 10240
EPT = E // NS
RDT = NPAD // NS


@functools.partial(
    pl.kernel,
    out_type=jax.ShapeDtypeStruct((N, D), _f32),
    mesh=_mesh,
    scratch_types=[
        pltpu.VMEM((EPT,), _i32),
        pltpu.VMEM((NPAD,), _f32),
        pltpu.VMEM((RDT,), _f32),
        pltpu.VMEM((K, D), _f32),
        pltpu.VMEM_SHARED((NS, NPAD), _f32),
    ],
    compiler_params=_sc_params,
)
def _sc_dinv(dst_hbm, out_hbm, didx, acc, dsum, stage, spbuf):
    c = lax.axis_index("c")
    s = lax.axis_index("s")

    @pl.when(c == 0)
    def _():
        def zero_body(i, carry):
            acc[pl.ds(i * 16, 16)] = jnp.zeros((16,), _f32)
            return carry

        lax.fori_loop(0, NPAD // 16, zero_body, 0)

        pltpu.sync_copy(dst_hbm.at[pl.ds(s * EPT, EPT)], didx)
        ones16 = jnp.ones((16,), _f32)

        def body(j, carry):
            idxv = didx[pl.ds(j * 16, 16)]
            plsc.addupdate_scatter(acc, [idxv], ones16)
            return carry

        lax.fori_loop(0, EPT // 16, body, 0)

        pltpu.sync_copy(acc, spbuf.at[s])
        plsc.subcore_barrier()
        for k in range(NS):
            pltpu.sync_copy(spbuf.at[k, pl.ds(s * RDT, RDT)], acc.at[pl.ds(0, RDT)])
            if k == 0:
                def cp_body(i, carry):
                    dsum[pl.ds(i * 16, 16)] = acc[pl.ds(i * 16, 16)]
                    return carry
                lax.fori_loop(0, RDT // 16, cp_body, 0)
            else:
                def add_body(i, carry):
                    dsum[pl.ds(i * 16, 16)] = (dsum[pl.ds(i * 16, 16)]
                                               + acc[pl.ds(i * 16, 16)])
                    return carry
                lax.fori_loop(0, RDT // 16, add_body, 0)

        def rsqrt_body(i, carry):
            deg = dsum[pl.ds(i * 16, 16)] + 1.0
            bits = plsc.bitcast(deg, _i32)
            y = plsc.bitcast(0x5F3759DF - lax.shift_right_logical(bits, 1), _f32)
            for _ in range(3):
                y = y * (1.5 - 0.5 * deg * y * y)
            dsum[pl.ds(i * 16, 16)] = y
            return carry

        lax.fori_loop(0, RDT // 16, rsqrt_body, 0)

        def row_body(r, carry):
            v = plsc.load_gather(dsum, [jnp.full((16,), r, _i32)])
            for u in range(D // 16):
                stage[r % K, pl.ds(u * 16, 16)] = v
            return carry

        nchunk_full = RDT // K
        for t in range(nchunk_full):
            lax.fori_loop(t * K, (t + 1) * K, row_body, 0)
            row0 = s * RDT + t * K
            @pl.when(row0 + K <= N)
            def _():
                pltpu.sync_copy(stage, out_hbm.at[pl.ds(row0, K)])

    plsc.subcore_barrier()


@functools.partial(
    pl.kernel,
    out_type=jax.ShapeDtypeStruct((NC, N, D), _f32),
    mesh=_mesh,
    scratch_types=[
        pltpu.VMEM((EPW,), _i32),
        pltpu.VMEM((K,), _i32),
        pltpu.VMEM((K,), _i32),
        pltpu.VMEM((K, D), _f32),
        pltpu.VMEM((K, D), _f32),
        pltpu.VMEM_SHARED((N, D), _f32),
        pltpu.SemaphoreType.DMA,
        pltpu.SemaphoreType.DMA,
        pltpu.SemaphoreType.DMA,
        pltpu.SemaphoreType.DMA,
    ],
    compiler_params=_sc_params,
)
def _sc_scatter_rows(y_hbm, src_hbm, dst_hbm, out_hbm,
                     sidx, didx_a, didx_b, rows0, rows1, acc,
                     sem0, sem1, sem_s0, sem_s1):
    c = lax.axis_index("c")
    s = lax.axis_index("s")
    wid = s * NC + c

    def zero_body(i, carry):
        for u in range(D // 16):
            rows0[i, pl.ds(u * 16, 16)] = jnp.zeros((16,), _f32)
        return carry

    lax.fori_loop(0, K, zero_body, 0)
    for t in range(RDT // K):
        row0 = s * RDT + t * K
        @pl.when(row0 + K <= N)
        def _():
            pltpu.sync_copy(rows0, acc.at[pl.ds(row0, K)])

    plsc.subcore_barrier()

    base = wid * EPW
    pltpu.sync_copy(src_hbm.at[pl.ds(base, EPW)], sidx)

    def sch(j):
        return sidx.at[pl.ds(j * K, K)]

    def gat(j, rows, sem):
        return pltpu.make_async_copy(y_hbm.at[sch(j)], rows, sem)

    def sct(rows, didx, sem):
        return pltpu.make_async_copy(rows, acc.at[didx], sem)

    pltpu.sync_copy(dst_hbm.at[pl.ds(base, K)], didx_a)
    pltpu.async_copy(y_hbm.at[sch(0)], rows0, sem0)
    pltpu.sync_copy(dst_hbm.at[pl.ds(base + K, K)], didx_b)
    pltpu.async_copy(y_hbm.at[sch(1)], rows1, sem1)

    def body(i, carry):
        j = 2 * i
        gat(j, rows0, sem0).wait()
        pltpu.async_copy(rows0, acc.at[didx_a], sem_s0, add=True)

        @pl.when(j + 1 < NCH)
        def _():
            gat(j + 1, rows1, sem1).wait()
            pltpu.async_copy(rows1, acc.at[didx_b], sem_s1, add=True)

        sct(rows0, didx_a, sem_s0).wait()

        @pl.when(j + 2 < NCH)
        def _():
            pltpu.async_copy(y_hbm.at[sch(j + 2)], rows0, sem0)
            pltpu.sync_copy(dst_hbm.at[pl.ds(base + (j + 2) * K, K)], didx_a)

        @pl.when(j + 1 < NCH)
        def _():
            sct(rows1, didx_b, sem_s1).wait()

        @pl.when(j + 3 < NCH)
        def _():
            pltpu.async_copy(y_hbm.at[sch(j + 3)], rows1, sem1)
            pltpu.sync_copy(dst_hbm.at[pl.ds(base + (j + 3) * K, K)], didx_b)

        return carry

    lax.fori_loop(0, (NCH + 1) // 2, body, 0)
    plsc.subcore_barrier()

    for t in range(RDT // K):
        row0 = s * RDT + t * K
        @pl.when(row0 + K <= N)
        def _():
            pltpu.sync_copy(acc.at[pl.ds(row0, K)], rows0)
            pltpu.sync_copy(rows0, out_hbm.at[c, pl.ds(row0, K)])

@functools.partial(
    pl.kernel,
    out_type=jax.ShapeDtypeStruct((E,), _f32),
    mesh=_mesh,
    scratch_types=[
        pltpu.VMEM((EPW,), _i32),
        pltpu.VMEM((EPW,), _i32),
        pltpu.VMEM((K, D), _f32),
        pltpu.VMEM((K, D), _f32),
        pltpu.VMEM((K, D), _f32),
        pltpu.VMEM((K, D), _f32),
        pltpu.VMEM((K, D), _f32),
        pltpu.VMEM((K, D), _f32),
        pltpu.VMEM((K, D), _f32),
        pltpu.VMEM((K, D), _f32),
        pltpu.VMEM((EPW,), _f32),
        pltpu.SemaphoreType.DMA,
        pltpu.SemaphoreType.DMA,
        pltpu.SemaphoreType.DMA,
        pltpu.SemaphoreType.DMA,
    ],
    compiler_params=_sc_params,
)
def _sc_edge_dot(h_hbm, a_hbm, b_hbm, out_hbm,
                 aidx, bidx, ra0, rb0, ra1, rb1, ra2, rb2, ra3, rb3, outs,
                 sem0, sem1, sem2, sem3):
    c = lax.axis_index("c")
    s = lax.axis_index("s")
    wid = s * NC + c
    base = wid * EPW
    lanes = lax.iota(_i32, 16)

    pltpu.sync_copy(a_hbm.at[pl.ds(base, EPW)], aidx)
    pltpu.sync_copy(b_hbm.at[pl.ds(base, EPW)], bidx)

    bufs = ((ra0, rb0, sem0), (ra1, rb1, sem1), (ra2, rb2, sem2),
            (ra3, rb3, sem3))

    def gather_pair(j, ra, rb, sem):
        pltpu.async_copy(h_hbm.at[aidx.at[pl.ds(j * K, K)]], ra, sem)
        pltpu.async_copy(h_hbm.at[bidx.at[pl.ds(j * K, K)]], rb, sem)

    def wait_pair(j, ra, rb, sem):
        pltpu.make_async_copy(h_hbm.at[aidx.at[pl.ds(j * K, K)]], ra, sem).wait()
        pltpu.make_async_copy(h_hbm.at[bidx.at[pl.ds(j * K, K)]], rb, sem).wait()

    def compute(j, ra, rb):
        def col_body(t, accs):
            res = list(accs)
            for u in range(8):
                col = (lanes + (t * 8 + u)) & (D - 1)
                for g in range(K // 16):
                    row_ids = g * 16 + lanes
                    va = plsc.load_gather(ra, [row_ids, col])
                    vb = plsc.load_gather(rb, [row_ids, col])
                    res[g] = res[g] + va * vb
            return tuple(res)

        accs = lax.fori_loop(0, D // 8, col_body,
                             tuple(jnp.zeros((16,), _f32) for _ in range(K // 16)))
        for g in range(K // 16):
            outs[pl.ds(j * K + g * 16, 16)] = accs[g]

    gather_pair(0, ra0, rb0, sem0)
    gather_pair(1, ra1, rb1, sem1)
    gather_pair(2, ra2, rb2, sem2)

    def body(i, carry):
        for u in range(4):
            j = 4 * i + u
            ra, rb, sem = bufs[u]
            nra, nrb, nsem = bufs[(u + 3) % 4]

            @pl.when(j < NCH)
            def _():
                wait_pair(j, ra, rb, sem)

                @pl.when(j + 3 < NCH)
                def _():
                    gather_pair(j + 3, nra, nrb, nsem)

                compute(j, ra, rb)

        return carry

    lax.fori_loop(0, (NCH + 3) // 4, body, 0)
    pltpu.sync_copy(outs, out_hbm.at[pl.ds(base, EPW)])


_BLK = 1000


def _tc_first_body(x_ref, w_ref, dm_ref, y_ref):
    y_ref[...] = jnp.dot(x_ref[...], w_ref[...],
                         preferred_element_type=_f32) * dm_ref[...]


def _tc_first(x, W1, dm):
    return pl.pallas_call(
        _tc_first_body,
        grid=(N // _BLK,),
        in_specs=[
            pl.BlockSpec((_BLK, D), lambda i: (i, 0)),
            pl.BlockSpec((D, D), lambda i: (0, 0)),
            pl.BlockSpec((_BLK, D), lambda i: (i, 0)),
        ],
        out_specs=pl.BlockSpec((_BLK, D), lambda i: (i, 0)),
        out_shape=jax.ShapeDtypeStruct((N, D), _f32),
    )(x, W1, dm)


def _tc_mid_body(p_ref, y1_ref, dm_ref, b1_ref, w2_ref, y2_ref):
    dm = dm_ref[...]
    h = jax.nn.relu(dm * (p_ref[0] + p_ref[1] + y1_ref[...]) + b1_ref[...])
    y2_ref[...] = jnp.dot(h, w2_ref[...], preferred_element_type=_f32) * dm


def _tc_mid(p, y1, dm, b1, W2):
    return pl.pallas_call(
        _tc_mid_body,
        grid=(N // _BLK,),
        in_specs=[
            pl.BlockSpec((NC, _BLK, D), lambda i: (0, i, 0)),
            pl.BlockSpec((_BLK, D), lambda i: (i, 0)),
            pl.BlockSpec((_BLK, D), lambda i: (i, 0)),
            pl.BlockSpec((D,), lambda i: (0,)),
            pl.BlockSpec((D, D), lambda i: (0, 0)),
        ],
        out_specs=pl.BlockSpec((_BLK, D), lambda i: (i, 0)),
        out_shape=jax.ShapeDtypeStruct((N, D), _f32),
    )(p, y1, dm, b1, W2)


def _tc_last_body(q_ref, y2_ref, dm_ref, b2_ref, h2_ref):
    h2_ref[...] = dm_ref[...] * (q_ref[0] + q_ref[1] + y2_ref[...]) + b2_ref[...]


def _tc_last(q, y2, dm, b2):
    return pl.pallas_call(
        _tc_last_body,
        grid=(N // _BLK,),
        in_specs=[
            pl.BlockSpec((NC, _BLK, D), lambda i: (0, i, 0)),
            pl.BlockSpec((_BLK, D), lambda i: (i, 0)),
            pl.BlockSpec((_BLK, D), lambda i: (i, 0)),
            pl.BlockSpec((D,), lambda i: (0,)),
        ],
        out_specs=pl.BlockSpec((_BLK, D), lambda i: (i, 0)),
        out_shape=jax.ShapeDtypeStruct((N, D), _f32),
    )(q, y2, dm, b2)


def kernel(node_feature, edge_index, edge_label_index, W1, b1, W2, b2):
    src = edge_index[0]
    dst = edge_index[1]
    a = edge_label_index[0]
    b = edge_label_index[1]

    dm = _sc_dinv(dst)
    y1 = _tc_first(node_feature, W1, dm)
    p = _sc_scatter_rows(y1, src, dst)
    y2 = _tc_mid(p, y1, dm, b1, W2)
    q = _sc_scatter_rows(y2, src, dst)
    h2 = _tc_last(q, y2, dm, b2)
    return _sc_edge_dot(h2, a, b)


# speedup vs baseline: 1.0605x; 1.0605x over previous
import functools

import jax
import jax.numpy as jnp
from jax import lax
from jax.experimental import pallas as pl
from jax.experimental.pallas import tpu as pltpu
from jax.experimental.pallas import tpu_sc as plsc

N = 10000
D = 128
E = 320000

NC = 2
NS = 16
NW = NC * NS
EPW = E // NW
K = 80
NCH = EPW // K
ZW = 10
ZRPT = N // ZW

_mesh = plsc.VectorSubcoreMesh(core_axis_name="c", subcore_axis_name="s")
_sc_params = pltpu.CompilerParams(needs_layout_passes=False)
_f32 = jnp.float32
_i32 = jnp.int32


NPAD = 10240
EPT = E // NS
RDT = NPAD // NS


@functools.partial(
    pl.kernel,
    out_type=jax.ShapeDtypeStruct((N, D), _f32),
    mesh=_mesh,
    scratch_types=[
        pltpu.VMEM((EPT,), _i32),
        pltpu.VMEM((NPAD,), _f32),
        pltpu.VMEM((RDT,), _f32),
        pltpu.VMEM((K, D), _f32),
        pltpu.VMEM_SHARED((NS, NPAD), _f32),
    ],
    compiler_params=_sc_params,
)
def _sc_dinv(dst_hbm, out_hbm, didx, acc, dsum, stage, spbuf):
    c = lax.axis_index("c")
    s = lax.axis_index("s")

    @pl.when(c == 0)
    def _():
        def zero_body(i, carry):
            acc[pl.ds(i * 16, 16)] = jnp.zeros((16,), _f32)
            return carry

        lax.fori_loop(0, NPAD // 16, zero_body, 0)

        pltpu.sync_copy(dst_hbm.at[pl.ds(s * EPT, EPT)], didx)
        ones16 = jnp.ones((16,), _f32)

        def body(j, carry):
            idxv = didx[pl.ds(j * 16, 16)]
            plsc.addupdate_scatter(acc, [idxv], ones16)
            return carry

        lax.fori_loop(0, EPT // 16, body, 0)

        pltpu.sync_copy(acc, spbuf.at[s])
        plsc.subcore_barrier()
        for k in range(NS):
            pltpu.sync_copy(spbuf.at[k, pl.ds(s * RDT, RDT)], acc.at[pl.ds(0, RDT)])
            if k == 0:
                def cp_body(i, carry):
                    dsum[pl.ds(i * 16, 16)] = acc[pl.ds(i * 16, 16)]
                    return carry
                lax.fori_loop(0, RDT // 16, cp_body, 0)
            else:
                def add_body(i, carry):
                    dsum[pl.ds(i * 16, 16)] = (dsum[pl.ds(i * 16, 16)]
                                               + acc[pl.ds(i * 16, 16)])
                    return carry
                lax.fori_loop(0, RDT // 16, add_body, 0)

        def rsqrt_body(i, carry):
            deg = dsum[pl.ds(i * 16, 16)] + 1.0
            bits = plsc.bitcast(deg, _i32)
            y = plsc.bitcast(0x5F3759DF - lax.shift_right_logical(bits, 1), _f32)
            for _ in range(3):
                y = y * (1.5 - 0.5 * deg * y * y)
            dsum[pl.ds(i * 16, 16)] = y
            return carry

        lax.fori_loop(0, RDT // 16, rsqrt_body, 0)

        def row_body(r, carry):
            v = plsc.load_gather(dsum, [jnp.full((16,), r, _i32)])
            for u in range(D // 16):
                stage[r % K, pl.ds(u * 16, 16)] = v
            return carry

        nchunk_full = RDT // K
        for t in range(nchunk_full):
            lax.fori_loop(t * K, (t + 1) * K, row_body, 0)
            row0 = s * RDT + t * K
            @pl.when(row0 + K <= N)
            def _():
                pltpu.sync_copy(stage, out_hbm.at[pl.ds(row0, K)])

    plsc.subcore_barrier()


@functools.partial(
    pl.kernel,
    out_type=jax.ShapeDtypeStruct((NC, N, D), _f32),
    mesh=_mesh,
    scratch_types=[
        pltpu.VMEM((EPW,), _i32),
        pltpu.VMEM((K,), _i32),
        pltpu.VMEM((K,), _i32),
        pltpu.VMEM((K, D), _f32),
        pltpu.VMEM((K, D), _f32),
        pltpu.VMEM_SHARED((N, D), _f32),
        pltpu.SemaphoreType.DMA,
        pltpu.SemaphoreType.DMA,
        pltpu.SemaphoreType.DMA,
        pltpu.SemaphoreType.DMA,
    ],
    compiler_params=_sc_params,
)
def _sc_scatter_rows(y_hbm, src_hbm, dst_hbm, out_hbm,
                     sidx, didx_a, didx_b, rows0, rows1, acc,
                     sem0, sem1, sem_s0, sem_s1):
    c = lax.axis_index("c")
    s = lax.axis_index("s")
    wid = s * NC + c

    def zero_body(i, carry):
        for u in range(D // 16):
            rows0[i, pl.ds(u * 16, 16)] = jnp.zeros((16,), _f32)
        return carry

    lax.fori_loop(0, K, zero_body, 0)
    for t in range(RDT // K):
        row0 = s * RDT + t * K
        @pl.when(row0 + K <= N)
        def _():
            pltpu.sync_copy(rows0, acc.at[pl.ds(row0, K)])

    plsc.subcore_barrier()

    base = wid * EPW
    pltpu.sync_copy(src_hbm.at[pl.ds(base, EPW)], sidx)

    def sch(j):
        return sidx.at[pl.ds(j * K, K)]

    def gat(j, rows, sem):
        return pltpu.make_async_copy(y_hbm.at[sch(j)], rows, sem)

    def sct(rows, didx, sem):
        return pltpu.make_async_copy(rows, acc.at[didx], sem)

    pltpu.sync_copy(dst_hbm.at[pl.ds(base, K)], didx_a)
    pltpu.async_copy(y_hbm.at[sch(0)], rows0, sem0)
    pltpu.sync_copy(dst_hbm.at[pl.ds(base + K, K)], didx_b)
    pltpu.async_copy(y_hbm.at[sch(1)], rows1, sem1)

    def body(i, carry):
        j = 2 * i
        gat(j, rows0, sem0).wait()
        pltpu.async_copy(rows0, acc.at[didx_a], sem_s0, add=True)

        @pl.when(j + 1 < NCH)
        def _():
            gat(j + 1, rows1, sem1).wait()
            pltpu.async_copy(rows1, acc.at[didx_b], sem_s1, add=True)

        sct(rows0, didx_a, sem_s0).wait()

        @pl.when(j + 2 < NCH)
        def _():
            pltpu.async_copy(y_hbm.at[sch(j + 2)], rows0, sem0)
            pltpu.sync_copy(dst_hbm.at[pl.ds(base + (j + 2) * K, K)], didx_a)

        @pl.when(j + 1 < NCH)
        def _():
            sct(rows1, didx_b, sem_s1).wait()

        @pl.when(j + 3 < NCH)
        def _():
            pltpu.async_copy(y_hbm.at[sch(j + 3)], rows1, sem1)
            pltpu.sync_copy(dst_hbm.at[pl.ds(base + (j + 3) * K, K)], didx_b)

        return carry

    lax.fori_loop(0, (NCH + 1) // 2, body, 0)
    plsc.subcore_barrier()

    for t in range(RDT // K):
        row0 = s * RDT + t * K
        @pl.when(row0 + K <= N)
        def _():
            pltpu.sync_copy(acc.at[pl.ds(row0, K)], rows0)
            pltpu.sync_copy(rows0, out_hbm.at[c, pl.ds(row0, K)])

@functools.partial(
    pl.kernel,
    out_type=jax.ShapeDtypeStruct((E,), _f32),
    mesh=_mesh,
    scratch_types=[
        pltpu.VMEM((EPW,), _i32),
        pltpu.VMEM((EPW,), _i32),
        pltpu.VMEM((K, D), _f32),
        pltpu.VMEM((K, D), _f32),
        pltpu.VMEM((K, D), _f32),
        pltpu.VMEM((K, D), _f32),
        pltpu.VMEM((K, D), _f32),
        pltpu.VMEM((K, D), _f32),
        pltpu.VMEM((K, D), _f32),
        pltpu.VMEM((K, D), _f32),
        pltpu.VMEM((EPW,), _f32),
        pltpu.SemaphoreType.DMA,
        pltpu.SemaphoreType.DMA,
        pltpu.SemaphoreType.DMA,
        pltpu.SemaphoreType.DMA,
    ],
    compiler_params=_sc_params,
)
def _sc_edge_dot(h_hbm, a_hbm, b_hbm, out_hbm,
                 aidx, bidx, ra0, rb0, ra1, rb1, ra2, rb2, ra3, rb3, outs,
                 sem0, sem1, sem2, sem3):
    c = lax.axis_index("c")
    s = lax.axis_index("s")
    wid = s * NC + c
    base = wid * EPW
    lanes = lax.iota(_i32, 16)

    pltpu.sync_copy(a_hbm.at[pl.ds(base, EPW)], aidx)
    pltpu.sync_copy(b_hbm.at[pl.ds(base, EPW)], bidx)

    bufs = ((ra0, rb0, sem0), (ra1, rb1, sem1), (ra2, rb2, sem2),
            (ra3, rb3, sem3))

    def gather_pair(j, ra, rb, sem):
        pltpu.async_copy(h_hbm.at[aidx.at[pl.ds(j * K, K)]], ra, sem)
        pltpu.async_copy(h_hbm.at[bidx.at[pl.ds(j * K, K)]], rb, sem)

    def wait_pair(j, ra, rb, sem):
        pltpu.make_async_copy(h_hbm.at[aidx.at[pl.ds(j * K, K)]], ra, sem).wait()
        pltpu.make_async_copy(h_hbm.at[bidx.at[pl.ds(j * K, K)]], rb, sem).wait()

    def compute(j, ra, rb):
        def col_body(t, accs):
            res = list(accs)
            for u in range(4):
                col = (lanes + (t * 4 + u)) & (D - 1)
                for g in range(K // 16):
                    row_ids = g * 16 + lanes
                    va = plsc.load_gather(ra, [row_ids, col])
                    vb = plsc.load_gather(rb, [row_ids, col])
                    res[g] = res[g] + va * vb
            return tuple(res)

        accs = lax.fori_loop(0, D // 4, col_body,
                             tuple(jnp.zeros((16,), _f32) for _ in range(K // 16)))
        for g in range(K // 16):
            outs[pl.ds(j * K + g * 16, 16)] = accs[g]

    gather_pair(0, ra0, rb0, sem0)
    gather_pair(1, ra1, rb1, sem1)
    gather_pair(2, ra2, rb2, sem2)

    def body(i, carry):
        for u in range(4):
            j = 4 * i + u
            ra, rb, sem = bufs[u]
            nra, nrb, nsem = bufs[(u + 3) % 4]

            @pl.when(j < NCH)
            def _():
                wait_pair(j, ra, rb, sem)

                @pl.when(j + 3 < NCH)
                def _():
                    gather_pair(j + 3, nra, nrb, nsem)

                compute(j, ra, rb)

        return carry

    lax.fori_loop(0, (NCH + 3) // 4, body, 0)
    pltpu.sync_copy(outs, out_hbm.at[pl.ds(base, EPW)])


_BLK = 1000


def _tc_first_body(x_ref, w_ref, dm_ref, y_ref):
    y_ref[...] = jnp.dot(x_ref[...], w_ref[...],
                         preferred_element_type=_f32) * dm_ref[...]


def _tc_first(x, W1, dm):
    return pl.pallas_call(
        _tc_first_body,
        grid=(N // _BLK,),
        in_specs=[
            pl.BlockSpec((_BLK, D), lambda i: (i, 0)),
            pl.BlockSpec((D, D), lambda i: (0, 0)),
            pl.BlockSpec((_BLK, D), lambda i: (i, 0)),
        ],
        out_specs=pl.BlockSpec((_BLK, D), lambda i: (i, 0)),
        out_shape=jax.ShapeDtypeStruct((N, D), _f32),
    )(x, W1, dm)


def _tc_mid_body(p_ref, y1_ref, dm_ref, b1_ref, w2_ref, y2_ref):
    dm = dm_ref[...]
    h = jax.nn.relu(dm * (p_ref[0] + p_ref[1] + y1_ref[...]) + b1_ref[...])
    y2_ref[...] = jnp.dot(h, w2_ref[...], preferred_element_type=_f32) * dm


def _tc_mid(p, y1, dm, b1, W2):
    return pl.pallas_call(
        _tc_mid_body,
        grid=(N // _BLK,),
        in_specs=[
            pl.BlockSpec((NC, _BLK, D), lambda i: (0, i, 0)),
            pl.BlockSpec((_BLK, D), lambda i: (i, 0)),
            pl.BlockSpec((_BLK, D), lambda i: (i, 0)),
            pl.BlockSpec((D,), lambda i: (0,)),
            pl.BlockSpec((D, D), lambda i: (0, 0)),
        ],
        out_specs=pl.BlockSpec((_BLK, D), lambda i: (i, 0)),
        out_shape=jax.ShapeDtypeStruct((N, D), _f32),
    )(p, y1, dm, b1, W2)


def _tc_last_body(q_ref, y2_ref, dm_ref, b2_ref, h2_ref):
    h2_ref[...] = dm_ref[...] * (q_ref[0] + q_ref[1] + y2_ref[...]) + b2_ref[...]


def _tc_last(q, y2, dm, b2):
    return pl.pallas_call(
        _tc_last_body,
        grid=(N // _BLK,),
        in_specs=[
            pl.BlockSpec((NC, _BLK, D), lambda i: (0, i, 0)),
            pl.BlockSpec((_BLK, D), lambda i: (i, 0)),
            pl.BlockSpec((_BLK, D), lambda i: (i, 0)),
            pl.BlockSpec((D,), lambda i: (0,)),
        ],
        out_specs=pl.BlockSpec((_BLK, D), lambda i: (i, 0)),
        out_shape=jax.ShapeDtypeStruct((N, D), _f32),
    )(q, y2, dm, b2)


def kernel(node_feature, edge_index, edge_label_index, W1, b1, W2, b2):
    src = edge_index[0]
    dst = edge_index[1]
    a = edge_label_index[0]
    b = edge_label_index[1]

    dm = _sc_dinv(dst)
    y1 = _tc_first(node_feature, W1, dm)
    p = _sc_scatter_rows(y1, src, dst)
    y2 = _tc_mid(p, y1, dm, b1, W2)
    q = _sc_scatter_rows(y2, src, dst)
    h2 = _tc_last(q, y2, dm, b2)
    return _sc_edge_dot(h2, a, b)


# speedup vs baseline: 1.0623x; 1.0017x over previous
import functools

import jax
import jax.numpy as jnp
from jax import lax
from jax.experimental import pallas as pl
from jax.experimental.pallas import tpu as pltpu
from jax.experimental.pallas import tpu_sc as plsc

N = 10000
D = 128
E = 320000

NC = 2
NS = 16
NW = NC * NS
EPW = E // NW
K = 80
NCH = EPW // K
_mesh = plsc.VectorSubcoreMesh(core_axis_name="c", subcore_axis_name="s")
_sc_params = pltpu.CompilerParams(needs_layout_passes=False)
_f32 = jnp.float32
_i32 = jnp.int32


NPAD = 10240
EPT = E // NS
RDT = NPAD // NS


@functools.partial(
    pl.kernel,
    out_type=jax.ShapeDtypeStruct((N, D), _f32),
    mesh=_mesh,
    scratch_types=[
        pltpu.VMEM((EPT,), _i32),
        pltpu.VMEM((NPAD,), _f32),
        pltpu.VMEM((RDT,), _f32),
        pltpu.VMEM((K, D), _f32),
        pltpu.VMEM_SHARED((NS, NPAD), _f32),
    ],
    compiler_params=_sc_params,
)
def _sc_dinv(dst_hbm, out_hbm, didx, acc, dsum, stage, spbuf):
    c = lax.axis_index("c")
    s = lax.axis_index("s")

    @pl.when(c == 0)
    def _():
        def zero_body(i, carry):
            acc[pl.ds(i * 16, 16)] = jnp.zeros((16,), _f32)
            return carry

        lax.fori_loop(0, NPAD // 16, zero_body, 0)

        pltpu.sync_copy(dst_hbm.at[pl.ds(s * EPT, EPT)], didx)
        ones16 = jnp.ones((16,), _f32)

        def body(j, carry):
            idxv = didx[pl.ds(j * 16, 16)]
            plsc.addupdate_scatter(acc, [idxv], ones16)
            return carry

        lax.fori_loop(0, EPT // 16, body, 0)

        pltpu.sync_copy(acc, spbuf.at[s])
        plsc.subcore_barrier()
        for k in range(NS):
            pltpu.sync_copy(spbuf.at[k, pl.ds(s * RDT, RDT)], acc.at[pl.ds(0, RDT)])
            if k == 0:
                def cp_body(i, carry):
                    dsum[pl.ds(i * 16, 16)] = acc[pl.ds(i * 16, 16)]
                    return carry
                lax.fori_loop(0, RDT // 16, cp_body, 0)
            else:
                def add_body(i, carry):
                    dsum[pl.ds(i * 16, 16)] = (dsum[pl.ds(i * 16, 16)]
                                               + acc[pl.ds(i * 16, 16)])
                    return carry
                lax.fori_loop(0, RDT // 16, add_body, 0)

        def rsqrt_body(i, carry):
            deg = dsum[pl.ds(i * 16, 16)] + 1.0
            bits = plsc.bitcast(deg, _i32)
            y = plsc.bitcast(0x5F3759DF - lax.shift_right_logical(bits, 1), _f32)
            for _ in range(3):
                y = y * (1.5 - 0.5 * deg * y * y)
            dsum[pl.ds(i * 16, 16)] = y
            return carry

        lax.fori_loop(0, RDT // 16, rsqrt_body, 0)

        def row_body(r, carry):
            v = plsc.load_gather(dsum, [jnp.full((16,), r, _i32)])
            for u in range(D // 16):
                stage[r % K, pl.ds(u * 16, 16)] = v
            return carry

        nchunk_full = RDT // K
        for t in range(nchunk_full):
            lax.fori_loop(t * K, (t + 1) * K, row_body, 0)
            row0 = s * RDT + t * K
            @pl.when(row0 + K <= N)
            def _():
                pltpu.sync_copy(stage, out_hbm.at[pl.ds(row0, K)])

    plsc.subcore_barrier()


@functools.partial(
    pl.kernel,
    out_type=jax.ShapeDtypeStruct((NC, N, D), _f32),
    mesh=_mesh,
    scratch_types=[
        pltpu.VMEM((EPW,), _i32),
        pltpu.VMEM((K,), _i32),
        pltpu.VMEM((K,), _i32),
        pltpu.VMEM((K, D), _f32),
        pltpu.VMEM((K, D), _f32),
        pltpu.VMEM_SHARED((N, D), _f32),
        pltpu.SemaphoreType.DMA,
        pltpu.SemaphoreType.DMA,
        pltpu.SemaphoreType.DMA,
        pltpu.SemaphoreType.DMA,
    ],
    compiler_params=_sc_params,
)
def _sc_scatter_rows(y_hbm, src_hbm, dst_hbm, out_hbm,
                     sidx, didx_a, didx_b, rows0, rows1, acc,
                     sem0, sem1, sem_s0, sem_s1):
    c = lax.axis_index("c")
    s = lax.axis_index("s")
    wid = s * NC + c

    def zero_body(i, carry):
        for u in range(D // 16):
            rows0[i, pl.ds(u * 16, 16)] = jnp.zeros((16,), _f32)
        return carry

    lax.fori_loop(0, K, zero_body, 0)
    for t in range(RDT // K):
        row0 = s * RDT + t * K
        @pl.when(row0 + K <= N)
        def _():
            pltpu.sync_copy(rows0, acc.at[pl.ds(row0, K)])

    plsc.subcore_barrier()

    base = wid * EPW
    pltpu.sync_copy(src_hbm.at[pl.ds(base, EPW)], sidx)

    def sch(j):
        return sidx.at[pl.ds(j * K, K)]

    def gat(j, rows, sem):
        return pltpu.make_async_copy(y_hbm.at[sch(j)], rows, sem)

    def sct(rows, didx, sem):
        return pltpu.make_async_copy(rows, acc.at[didx], sem)

    pltpu.sync_copy(dst_hbm.at[pl.ds(base, K)], didx_a)
    pltpu.async_copy(y_hbm.at[sch(0)], rows0, sem0)
    pltpu.sync_copy(dst_hbm.at[pl.ds(base + K, K)], didx_b)
    pltpu.async_copy(y_hbm.at[sch(1)], rows1, sem1)

    def body(i, carry):
        j = 2 * i
        gat(j, rows0, sem0).wait()
        pltpu.async_copy(rows0, acc.at[didx_a], sem_s0, add=True)

        @pl.when(j + 1 < NCH)
        def _():
            gat(j + 1, rows1, sem1).wait()
            pltpu.async_copy(rows1, acc.at[didx_b], sem_s1, add=True)

        sct(rows0, didx_a, sem_s0).wait()

        @pl.when(j + 2 < NCH)
        def _():
            pltpu.async_copy(y_hbm.at[sch(j + 2)], rows0, sem0)
            pltpu.sync_copy(dst_hbm.at[pl.ds(base + (j + 2) * K, K)], didx_a)

        @pl.when(j + 1 < NCH)
        def _():
            sct(rows1, didx_b, sem_s1).wait()

        @pl.when(j + 3 < NCH)
        def _():
            pltpu.async_copy(y_hbm.at[sch(j + 3)], rows1, sem1)
            pltpu.sync_copy(dst_hbm.at[pl.ds(base + (j + 3) * K, K)], didx_b)

        return carry

    lax.fori_loop(0, (NCH + 1) // 2, body, 0)
    plsc.subcore_barrier()

    for t in range(RDT // K):
        row0 = s * RDT + t * K
        @pl.when(row0 + K <= N)
        def _():
            pltpu.sync_copy(acc.at[pl.ds(row0, K)], rows0)
            pltpu.sync_copy(rows0, out_hbm.at[c, pl.ds(row0, K)])

@functools.partial(
    pl.kernel,
    out_type=jax.ShapeDtypeStruct((E,), _f32),
    mesh=_mesh,
    scratch_types=[
        pltpu.VMEM((EPW,), _i32),
        pltpu.VMEM((EPW,), _i32),
        pltpu.VMEM((K, D), _f32),
        pltpu.VMEM((K, D), _f32),
        pltpu.VMEM((K, D), _f32),
        pltpu.VMEM((K, D), _f32),
        pltpu.VMEM((K, D), _f32),
        pltpu.VMEM((K, D), _f32),
        pltpu.VMEM((K, D), _f32),
        pltpu.VMEM((K, D), _f32),
        pltpu.VMEM((EPW,), _f32),
        pltpu.SemaphoreType.DMA,
        pltpu.SemaphoreType.DMA,
        pltpu.SemaphoreType.DMA,
        pltpu.SemaphoreType.DMA,
    ],
    compiler_params=_sc_params,
)
def _sc_edge_dot(h_hbm, a_hbm, b_hbm, out_hbm,
                 aidx, bidx, ra0, rb0, ra1, rb1, ra2, rb2, ra3, rb3, outs,
                 sem0, sem1, sem2, sem3):
    c = lax.axis_index("c")
    s = lax.axis_index("s")
    wid = s * NC + c
    base = wid * EPW
    lanes = lax.iota(_i32, 16)

    pltpu.sync_copy(a_hbm.at[pl.ds(base, EPW)], aidx)
    pltpu.sync_copy(b_hbm.at[pl.ds(base, EPW)], bidx)

    bufs = ((ra0, rb0, sem0), (ra1, rb1, sem1), (ra2, rb2, sem2),
            (ra3, rb3, sem3))

    def gather_pair(j, ra, rb, sem):
        pltpu.async_copy(h_hbm.at[aidx.at[pl.ds(j * K, K)]], ra, sem)
        pltpu.async_copy(h_hbm.at[bidx.at[pl.ds(j * K, K)]], rb, sem)

    def wait_pair(j, ra, rb, sem):
        pltpu.make_async_copy(h_hbm.at[aidx.at[pl.ds(j * K, K)]], ra, sem).wait()
        pltpu.make_async_copy(h_hbm.at[bidx.at[pl.ds(j * K, K)]], rb, sem).wait()

    def compute(j, ra, rb):
        def col_body(t, accs):
            res = list(accs)
            for u in range(4):
                col = (lanes + (t * 4 + u)) & (D - 1)
                for g in range(K // 16):
                    row_ids = g * 16 + lanes
                    va = plsc.load_gather(ra, [row_ids, col])
                    vb = plsc.load_gather(rb, [row_ids, col])
                    res[g] = res[g] + va * vb
            return tuple(res)

        accs = lax.fori_loop(0, D // 4, col_body,
                             tuple(jnp.zeros((16,), _f32) for _ in range(K // 16)))
        for g in range(K // 16):
            outs[pl.ds(j * K + g * 16, 16)] = accs[g]

    gather_pair(0, ra0, rb0, sem0)
    gather_pair(1, ra1, rb1, sem1)
    gather_pair(2, ra2, rb2, sem2)

    def body(i, carry):
        for u in range(4):
            j = 4 * i + u
            ra, rb, sem = bufs[u]
            nra, nrb, nsem = bufs[(u + 3) % 4]

            @pl.when(j < NCH)
            def _():
                wait_pair(j, ra, rb, sem)

                @pl.when(j + 3 < NCH)
                def _():
                    gather_pair(j + 3, nra, nrb, nsem)

                compute(j, ra, rb)

        return carry

    lax.fori_loop(0, (NCH + 3) // 4, body, 0)
    pltpu.sync_copy(outs, out_hbm.at[pl.ds(base, EPW)])


_BLK = 1000


def _tc_first_body(x_ref, w_ref, dm_ref, y_ref):
    y_ref[...] = jnp.dot(x_ref[...], w_ref[...],
                         preferred_element_type=_f32) * dm_ref[...]


def _tc_first(x, W1, dm):
    return pl.pallas_call(
        _tc_first_body,
        grid=(N // _BLK,),
        in_specs=[
            pl.BlockSpec((_BLK, D), lambda i: (i, 0)),
            pl.BlockSpec((D, D), lambda i: (0, 0)),
            pl.BlockSpec((_BLK, D), lambda i: (i, 0)),
        ],
        out_specs=pl.BlockSpec((_BLK, D), lambda i: (i, 0)),
        out_shape=jax.ShapeDtypeStruct((N, D), _f32),
    )(x, W1, dm)


def _tc_mid_body(p_ref, y1_ref, dm_ref, b1_ref, w2_ref, y2_ref):
    dm = dm_ref[...]
    h = jax.nn.relu(dm * (p_ref[0] + p_ref[1] + y1_ref[...]) + b1_ref[...])
    y2_ref[...] = jnp.dot(h, w2_ref[...], preferred_element_type=_f32) * dm


def _tc_mid(p, y1, dm, b1, W2):
    return pl.pallas_call(
        _tc_mid_body,
        grid=(N // _BLK,),
        in_specs=[
            pl.BlockSpec((NC, _BLK, D), lambda i: (0, i, 0)),
            pl.BlockSpec((_BLK, D), lambda i: (i, 0)),
            pl.BlockSpec((_BLK, D), lambda i: (i, 0)),
            pl.BlockSpec((D,), lambda i: (0,)),
            pl.BlockSpec((D, D), lambda i: (0, 0)),
        ],
        out_specs=pl.BlockSpec((_BLK, D), lambda i: (i, 0)),
        out_shape=jax.ShapeDtypeStruct((N, D), _f32),
    )(p, y1, dm, b1, W2)


def _tc_last_body(q_ref, y2_ref, dm_ref, b2_ref, h2_ref):
    h2_ref[...] = dm_ref[...] * (q_ref[0] + q_ref[1] + y2_ref[...]) + b2_ref[...]


def _tc_last(q, y2, dm, b2):
    return pl.pallas_call(
        _tc_last_body,
        grid=(N // _BLK,),
        in_specs=[
            pl.BlockSpec((NC, _BLK, D), lambda i: (0, i, 0)),
            pl.BlockSpec((_BLK, D), lambda i: (i, 0)),
            pl.BlockSpec((_BLK, D), lambda i: (i, 0)),
            pl.BlockSpec((D,), lambda i: (0,)),
        ],
        out_specs=pl.BlockSpec((_BLK, D), lambda i: (i, 0)),
        out_shape=jax.ShapeDtypeStruct((N, D), _f32),
    )(q, y2, dm, b2)


def kernel(node_feature, edge_index, edge_label_index, W1, b1, W2, b2):
    src = edge_index[0]
    dst = edge_index[1]
    a = edge_label_index[0]
    b = edge_label_index[1]

    dm = _sc_dinv(dst)
    y1 = _tc_first(node_feature, W1, dm)
    p = _sc_scatter_rows(y1, src, dst)
    y2 = _tc_mid(p, y1, dm, b1, W2)
    q = _sc_scatter_rows(y2, src, dst)
    h2 = _tc_last(q, y2, dm, b2)
    return _sc_edge_dot(h2, a, b)
